# baseline (speedup 1.0000x reference)
import functools

import jax
import jax.numpy as jnp
from jax import lax
from jax.experimental import pallas as pl
from jax.experimental.pallas import tpu as pltpu
from jax.experimental.pallas import tpu_sc as plsc

B, N = 2, 262144
H, W = 384, 384
KS = 5
PAD = KS // 2
TOTAL = B * H * W

HW = H * W
NTILES = 16
PPT = HW // NTILES
CP = 16384
NB = N // CP
IMAX = jnp.iinfo(jnp.int32).max


def _ce(a, b):
    return jnp.minimum(a, b), jnp.maximum(a, b)


def _drop_min_max(work):
    s = len(work)

    def ce(i, j):
        work[i], work[j] = _ce(work[i], work[j])

    for i in range(0, s - 1, 2):
        ce(i, i + 1)
    evens = list(range(0, s, 2))
    odds = list(range(1, s, 2))
    if s % 2 == 1:
        odds.append(s - 1)
    for e in evens[1:]:
        ce(evens[0], e)
    for o in odds[:-1]:
        ce(o, odds[-1])
    lo, hi = evens[0], odds[-1]
    return [work[i] for i in range(s) if i != lo and i != hi]


def _median25(w):
    work = list(w[:14])
    nxt = 14
    while len(work) > 3:
        work = _drop_min_max(work)
        work.append(w[nxt])
        nxt += 1
    a, b, c = work
    return jnp.maximum(jnp.minimum(a, b), jnp.minimum(jnp.maximum(a, b), c))


def _reflect_pad2(x):
    x = jnp.concatenate(
        [x[2:3, :], x[1:2, :], x, x[H - 2:H - 1, :], x[H - 3:H - 2, :]], axis=0)
    x = jnp.concatenate(
        [x[:, 2:3], x[:, 1:2], x, x[:, W - 2:W - 1], x[:, W - 3:W - 2]], axis=1)
    return x


def _inpaint_kernel(img_ref, out_ref):
    hole = img_ref[0, 0] <= 0.0

    def one_round(_, chans):
        outs = []
        for c in chans:
            p = _reflect_pad2(c)
            wins = [p[i:i + H, j:j + W] for i in range(KS) for j in range(KS)]
            med = _median25(wins)
            outs.append(jnp.where(hole, med, c))
        return tuple(outs)

    chans = tuple(img_ref[0, c] for c in range(3))
    chans = lax.fori_loop(0, 10, one_round, chans, unroll=False)

    s = ((chans[0] > 0.0) | (chans[1] > 0.0) | (chans[2] > 0.0)).astype(jnp.float32)
    sp = jnp.pad(s, ((1, 1), (1, 1)))
    neigh = sum(sp[i:i + H, j:j + W] for i in range(3) for j in range(3))
    keep = (neigh >= 6.0).astype(jnp.float32)
    for c in range(3):
        out_ref[0, c] = chans[c] * keep


def _inpaint(image):
    return pl.pallas_call(
        _inpaint_kernel,
        grid=(B,),
        in_specs=[pl.BlockSpec((1, 3, H, W), lambda b: (b, 0, 0, 0))],
        out_specs=pl.BlockSpec((1, 3, H, W), lambda b: (b, 0, 0, 0)),
        out_shape=jax.ShapeDtypeStruct((B, 3, H, W), jnp.float32),
    )(image)


def _bf(x):
    return x.astype(jnp.bfloat16).astype(jnp.float32)


def _proj_kernel(K_ref, T_ref, cloud_ref, lin_ref, dep_ref):
    b = pl.program_id(0)
    c0, c1, c2, c3 = (_bf(cloud_ref[0, i]) for i in range(4))
    cc = [(_bf(T_ref[0, i, 0]) * c0 + _bf(T_ref[0, i, 1]) * c1)
          + (_bf(T_ref[0, i, 2]) * c2 + _bf(T_ref[0, i, 3]) * c3)
          for i in range(3)]
    ccb = [_bf(x) for x in cc]
    pr = [(_bf(K_ref[0, i, 0]) * ccb[0] + _bf(K_ref[0, i, 1]) * ccb[1])
          + _bf(K_ref[0, i, 2]) * ccb[2] for i in range(3)]
    u = jnp.clip(pr[0] / pr[2], 0.0, W - 1).astype(jnp.int32)
    v = jnp.clip(pr[1] / pr[2], 0.0, H - 1).astype(jnp.int32)
    lin_ref[0] = b * HW + v * W + u
    dep_ref[0] = cc[2].astype(jnp.int32)


def _project(cloud, K, T):
    c4 = cloud.reshape(B, 4, 512, 512)
    lin, dep = pl.pallas_call(
        _proj_kernel,
        grid=(B,),
        in_specs=[
            pl.BlockSpec((1, 3, 3), lambda b: (b, 0, 0), memory_space=pltpu.SMEM),
            pl.BlockSpec((1, 4, 4), lambda b: (b, 0, 0), memory_space=pltpu.SMEM),
            pl.BlockSpec((1, 4, 512, 512), lambda b: (b, 0, 0, 0)),
        ],
        out_specs=[pl.BlockSpec((1, 512, 512), lambda b: (b, 0, 0))] * 2,
        out_shape=[jax.ShapeDtypeStruct((B, 512, 512), jnp.int32)] * 2,
    )(K, T, c4)
    return lin.reshape(-1), dep.reshape(-1)


def _any16(m):
    return plsc.all_reduce_population_count(m)[0] > 0


def _zbuf_body(lin_hbm, dep_hbm, r0_hbm, r1_hbm, r2_hbm,
               img0_hbm, img1_hbm, img2_hbm,
               idxv, dv, minbuf, idbuf, idsafe, g0, g1, g2, sem):
    cid = lax.axis_index("c")
    sid = lax.axis_index("s")
    lo = cid * HW + sid * PPT
    hi = lo + PPT
    pt_base = cid * N

    def init_bufs(k, _):
        minbuf[pl.ds(k * 16, 16)] = jnp.full((16,), IMAX, jnp.int32)
        idbuf[pl.ds(k * 16, 16)] = jnp.full((16,), -1, jnp.int32)
        return 0
    lax.fori_loop(0, PPT // 16, init_bufs, 0)

    GS = 4

    def blk_body(blk, _):
        off = pt_base + blk * CP
        pltpu.sync_copy(lin_hbm.at[pl.ds(off, CP)], idxv)
        pltpu.sync_copy(dep_hbm.at[pl.ds(off, CP)], dv)

        def chunk(i, _):
            base = i * (16 * GS)
            idxs = [idxv[pl.ds(base + k * 16, 16)] for k in range(GS)]
            inms = [(ix >= lo) & (ix < hi) for ix in idxs]
            offs = [jnp.clip(ix - lo, 0, PPT - 1) for ix in idxs]
            ds_ = [dv[pl.ds(base + k * 16, 16)] for k in range(GS)]
            gids = [pt_base + blk * CP + base + k * 16
                    + jnp.arange(16, dtype=jnp.int32) for k in range(GS)]
            betters = []
            for k in range(GS):
                c_d = plsc.load_gather(minbuf, [offs[k]], mask=inms[k])
                c_i = plsc.load_gather(idbuf, [offs[k]], mask=inms[k])
                betters.append(inms[k] & ((ds_[k] < c_d)
                                          | ((ds_[k] == c_d) & (gids[k] > c_i))))

            def cond(nds):
                m = nds[0]
                for k in range(1, GS):
                    m = m | nds[k]
                return _any16(m)

            def body(nds):
                out = []
                for k in range(GS):
                    nd = nds[k]
                    plsc.store_scatter(minbuf, [offs[k]], ds_[k], mask=nd)
                    c_d = plsc.load_gather(minbuf, [offs[k]], mask=nd)
                    idw = nd & (ds_[k] == c_d)
                    plsc.store_scatter(idbuf, [offs[k]], gids[k], mask=idw)
                    c_i = plsc.load_gather(idbuf, [offs[k]], mask=nd)
                    out.append(nd & ((ds_[k] < c_d)
                                     | ((ds_[k] == c_d) & (gids[k] > c_i))))
                return tuple(out)
            lax.while_loop(cond, body, tuple(betters))
            return 0
        lax.fori_loop(0, CP // (16 * GS), chunk, 0)
        return 0
    lax.fori_loop(0, NB, blk_body, 0)

    def mk_safe(k, _):
        ids = idbuf[pl.ds(k * 16, 16)]
        spread = pt_base + k * 16 + jnp.arange(16, dtype=jnp.int32)
        idsafe[pl.ds(k * 16, 16)] = jnp.where(ids >= 0, ids, spread)
        return 0
    lax.fori_loop(0, PPT // 16, mk_safe, 0)

    def gblk(gi, _):
        base = gi * 128
        h0 = pltpu.async_copy(r0_hbm.at[idsafe.at[pl.ds(base, 128)]],
                              g0.at[pl.ds(base, 128)], sem)
        h1 = pltpu.async_copy(r1_hbm.at[idsafe.at[pl.ds(base, 128)]],
                              g1.at[pl.ds(base, 128)], sem)
        h2 = pltpu.async_copy(r2_hbm.at[idsafe.at[pl.ds(base, 128)]],
                              g2.at[pl.ds(base, 128)], sem)
        h0.wait()
        h1.wait()
        h2.wait()
        return 0
    lax.fori_loop(0, PPT // 128, gblk, 0)

    def fin(k, _):
        sl = pl.ds(k * 16, 16)
        valid = idbuf[sl] >= 0
        g0[sl] = jnp.where(valid, g0[sl], -0.001)
        g1[sl] = jnp.where(valid, g1[sl], -0.001)
        g2[sl] = jnp.where(valid, g2[sl], -0.001)
        return 0
    lax.fori_loop(0, PPT // 16, fin, 0)

    pltpu.sync_copy(g0, img0_hbm.at[pl.ds(lo, PPT)])
    pltpu.sync_copy(g1, img1_hbm.at[pl.ds(lo, PPT)])
    pltpu.sync_copy(g2, img2_hbm.at[pl.ds(lo, PPT)])


def _zbuffer(lin, dep, r0, r1, r2):
    run = pl.kernel(
        _zbuf_body,
        out_type=[jax.ShapeDtypeStruct((TOTAL,), jnp.float32)] * 3,
        mesh=plsc.VectorSubcoreMesh(core_axis_name="c", subcore_axis_name="s"),
        compiler_params=pltpu.CompilerParams(needs_layout_passes=False),
        scratch_types=[
            pltpu.VMEM((CP,), jnp.int32),
            pltpu.VMEM((CP,), jnp.int32),
            pltpu.VMEM((PPT,), jnp.int32),
            pltpu.VMEM((PPT,), jnp.int32),
            pltpu.VMEM((PPT,), jnp.int32),
            pltpu.VMEM((PPT,), jnp.float32),
            pltpu.VMEM((PPT,), jnp.float32),
            pltpu.VMEM((PPT,), jnp.float32),
            pltpu.SemaphoreType.DMA,
        ],
    )
    return run(lin, dep, r0, r1, r2)


def kernel(cloud, rgb_vec, K, T):
    lin, dep = _project(cloud, K, T)
    rgbp = jnp.transpose(rgb_vec, (1, 0, 2)).reshape(3, -1)
    img0, img1, img2 = _zbuffer(lin, dep, rgbp[0], rgbp[1], rgbp[2])
    image = jnp.stack([img0, img1, img2], 0).reshape(3, B, H, W)
    image = jnp.transpose(image, (1, 0, 2, 3))
    return _inpaint(image)


# speedup vs baseline: 1.6209x; 1.6209x over previous
import functools

import jax
import jax.numpy as jnp
from jax import lax
from jax.experimental import pallas as pl
from jax.experimental.pallas import tpu as pltpu
from jax.experimental.pallas import tpu_sc as plsc

B, N = 2, 262144
H, W = 384, 384
KS = 5
PAD = KS // 2
TOTAL = B * H * W

HW = H * W
NTILES = 16
PPT = HW // NTILES
CP = 16384
NB = N // CP
IMAX = jnp.iinfo(jnp.int32).max


def _ce(a, b):
    return jnp.minimum(a, b), jnp.maximum(a, b)


def _median25(w):
    work = list(w[:14])
    nxt = 14
    while len(work) > 3:
        s = len(work)
        for i in range(s - 1):
            work[i], work[i + 1] = _ce(work[i], work[i + 1])
        for i in range(s - 2, 0, -1):
            work[i - 1], work[i] = _ce(work[i - 1], work[i])
        work = work[1:-1]
        work.append(w[nxt])
        nxt += 1
    a, b, c = work
    return jnp.maximum(jnp.minimum(a, b), jnp.minimum(jnp.maximum(a, b), c))


def _reflect_pad2(x):
    x = jnp.concatenate(
        [x[2:3, :], x[1:2, :], x, x[H - 2:H - 1, :], x[H - 3:H - 2, :]], axis=0)
    x = jnp.concatenate(
        [x[:, 2:3], x[:, 1:2], x, x[:, W - 2:W - 1], x[:, W - 3:W - 2]], axis=1)
    return x


def _inpaint_kernel(img_ref, out_ref):
    hole = img_ref[0, 0] <= 0.0

    def one_round(_, chans):
        outs = []
        for c in chans:
            p = _reflect_pad2(c)
            wins = [p[i:i + H, j:j + W] for i in range(KS) for j in range(KS)]
            med = _median25(wins)
            outs.append(jnp.where(hole, med, c))
        return tuple(outs)

    chans = tuple(img_ref[0, c] for c in range(3))
    chans = lax.fori_loop(0, 10, one_round, chans, unroll=False)

    s = ((chans[0] > 0.0) | (chans[1] > 0.0) | (chans[2] > 0.0)).astype(jnp.float32)
    sp = jnp.pad(s, ((1, 1), (1, 1)))
    neigh = sum(sp[i:i + H, j:j + W] for i in range(3) for j in range(3))
    keep = (neigh >= 6.0).astype(jnp.float32)
    for c in range(3):
        out_ref[0, c] = chans[c] * keep


def _inpaint(image):
    return pl.pallas_call(
        _inpaint_kernel,
        grid=(B,),
        in_specs=[pl.BlockSpec((1, 3, H, W), lambda b: (b, 0, 0, 0))],
        out_specs=pl.BlockSpec((1, 3, H, W), lambda b: (b, 0, 0, 0)),
        out_shape=jax.ShapeDtypeStruct((B, 3, H, W), jnp.float32),
    )(image)


def _bf(x):
    return x.astype(jnp.bfloat16).astype(jnp.float32)


def _proj_kernel(K_ref, T_ref, cloud_ref, lin_ref, dep_ref):
    b = pl.program_id(0)
    c0, c1, c2, c3 = (_bf(cloud_ref[0, i]) for i in range(4))
    cc = [(_bf(T_ref[0, i, 0]) * c0 + _bf(T_ref[0, i, 1]) * c1)
          + (_bf(T_ref[0, i, 2]) * c2 + _bf(T_ref[0, i, 3]) * c3)
          for i in range(3)]
    ccb = [_bf(x) for x in cc]
    pr = [(_bf(K_ref[0, i, 0]) * ccb[0] + _bf(K_ref[0, i, 1]) * ccb[1])
          + _bf(K_ref[0, i, 2]) * ccb[2] for i in range(3)]
    u = jnp.clip(pr[0] / pr[2], 0.0, W - 1).astype(jnp.int32)
    v = jnp.clip(pr[1] / pr[2], 0.0, H - 1).astype(jnp.int32)
    lin_ref[0] = b * HW + v * W + u
    dep_ref[0] = cc[2].astype(jnp.int32)


def _project(cloud, K, T):
    c4 = cloud.reshape(B, 4, 512, 512)
    lin, dep = pl.pallas_call(
        _proj_kernel,
        grid=(B,),
        in_specs=[
            pl.BlockSpec((1, 3, 3), lambda b: (b, 0, 0), memory_space=pltpu.SMEM),
            pl.BlockSpec((1, 4, 4), lambda b: (b, 0, 0), memory_space=pltpu.SMEM),
            pl.BlockSpec((1, 4, 512, 512), lambda b: (b, 0, 0, 0)),
        ],
        out_specs=[pl.BlockSpec((1, 512, 512), lambda b: (b, 0, 0))] * 2,
        out_shape=[jax.ShapeDtypeStruct((B, 512, 512), jnp.int32)] * 2,
    )(K, T, c4)
    return lin.reshape(-1), dep.reshape(-1)


def _any16(m):
    return plsc.all_reduce_population_count(m)[0] > 0


def _zbuf_body(lin_hbm, dep_hbm, r0_hbm, r1_hbm, r2_hbm,
               img0_hbm, img1_hbm, img2_hbm,
               idxv, dv, minbuf, idbuf, idsafe, g0, g1, g2, sem):
    cid = lax.axis_index("c")
    sid = lax.axis_index("s")
    lo = cid * HW + sid * PPT
    hi = lo + PPT
    pt_base = cid * N

    def init_bufs(k, _):
        minbuf[pl.ds(k * 16, 16)] = jnp.full((16,), IMAX, jnp.int32)
        idbuf[pl.ds(k * 16, 16)] = jnp.full((16,), -1, jnp.int32)
        return 0
    lax.fori_loop(0, PPT // 16, init_bufs, 0)

    GS = 4

    def blk_body(blk, _):
        off = pt_base + blk * CP
        pltpu.sync_copy(lin_hbm.at[pl.ds(off, CP)], idxv)
        pltpu.sync_copy(dep_hbm.at[pl.ds(off, CP)], dv)

        def chunk(i, _):
            base = i * (16 * GS)
            idxs = [idxv[pl.ds(base + k * 16, 16)] for k in range(GS)]
            inms = [(ix >= lo) & (ix < hi) for ix in idxs]
            offs = [jnp.clip(ix - lo, 0, PPT - 1) for ix in idxs]
            ds_ = [dv[pl.ds(base + k * 16, 16)] for k in range(GS)]
            gids = [pt_base + blk * CP + base + k * 16
                    + jnp.arange(16, dtype=jnp.int32) for k in range(GS)]
            betters = []
            for k in range(GS):
                c_d = plsc.load_gather(minbuf, [offs[k]], mask=inms[k])
                c_i = plsc.load_gather(idbuf, [offs[k]], mask=inms[k])
                betters.append(inms[k] & ((ds_[k] < c_d)
                                          | ((ds_[k] == c_d) & (gids[k] > c_i))))

            def cond(nds):
                m = nds[0]
                for k in range(1, GS):
                    m = m | nds[k]
                return _any16(m)

            def body(nds):
                out = []
                for k in range(GS):
                    nd = nds[k]
                    plsc.store_scatter(minbuf, [offs[k]], ds_[k], mask=nd)
                    c_d = plsc.load_gather(minbuf, [offs[k]], mask=nd)
                    idw = nd & (ds_[k] == c_d)
                    plsc.store_scatter(idbuf, [offs[k]], gids[k], mask=idw)
                    c_i = plsc.load_gather(idbuf, [offs[k]], mask=nd)
                    out.append(nd & ((ds_[k] < c_d)
                                     | ((ds_[k] == c_d) & (gids[k] > c_i))))
                return tuple(out)
            lax.while_loop(cond, body, tuple(betters))
            return 0
        lax.fori_loop(0, CP // (16 * GS), chunk, 0)
        return 0
    lax.fori_loop(0, NB, blk_body, 0)

    def mk_safe(k, _):
        ids = idbuf[pl.ds(k * 16, 16)]
        spread = pt_base + k * 16 + jnp.arange(16, dtype=jnp.int32)
        idsafe[pl.ds(k * 16, 16)] = jnp.where(ids >= 0, ids, spread)
        return 0
    lax.fori_loop(0, PPT // 16, mk_safe, 0)

    def gblk(gi, _):
        base = gi * 128
        h0 = pltpu.async_copy(r0_hbm.at[idsafe.at[pl.ds(base, 128)]],
                              g0.at[pl.ds(base, 128)], sem)
        h1 = pltpu.async_copy(r1_hbm.at[idsafe.at[pl.ds(base, 128)]],
                              g1.at[pl.ds(base, 128)], sem)
        h2 = pltpu.async_copy(r2_hbm.at[idsafe.at[pl.ds(base, 128)]],
                              g2.at[pl.ds(base, 128)], sem)
        h0.wait()
        h1.wait()
        h2.wait()
        return 0
    lax.fori_loop(0, PPT // 128, gblk, 0)

    def fin(k, _):
        sl = pl.ds(k * 16, 16)
        valid = idbuf[sl] >= 0
        g0[sl] = jnp.where(valid, g0[sl], -0.001)
        g1[sl] = jnp.where(valid, g1[sl], -0.001)
        g2[sl] = jnp.where(valid, g2[sl], -0.001)
        return 0
    lax.fori_loop(0, PPT // 16, fin, 0)

    pltpu.sync_copy(g0, img0_hbm.at[pl.ds(lo, PPT)])
    pltpu.sync_copy(g1, img1_hbm.at[pl.ds(lo, PPT)])
    pltpu.sync_copy(g2, img2_hbm.at[pl.ds(lo, PPT)])


def _zbuffer(lin, dep, r0, r1, r2):
    run = pl.kernel(
        _zbuf_body,
        out_type=[jax.ShapeDtypeStruct((TOTAL,), jnp.float32)] * 3,
        mesh=plsc.VectorSubcoreMesh(core_axis_name="c", subcore_axis_name="s"),
        compiler_params=pltpu.CompilerParams(needs_layout_passes=False),
        scratch_types=[
            pltpu.VMEM((CP,), jnp.int32),
            pltpu.VMEM((CP,), jnp.int32),
            pltpu.VMEM((PPT,), jnp.int32),
            pltpu.VMEM((PPT,), jnp.int32),
            pltpu.VMEM((PPT,), jnp.int32),
            pltpu.VMEM((PPT,), jnp.float32),
            pltpu.VMEM((PPT,), jnp.float32),
            pltpu.VMEM((PPT,), jnp.float32),
            pltpu.SemaphoreType.DMA,
        ],
    )
    return run(lin, dep, r0, r1, r2)


def kernel(cloud, rgb_vec, K, T):
    lin, dep = _project(cloud, K, T)
    rgbp = jnp.transpose(rgb_vec, (1, 0, 2)).reshape(3, -1)
    img0, img1, img2 = _zbuffer(lin, dep, rgbp[0], rgbp[1], rgbp[2])
    image = jnp.stack([img0, img1, img2], 0).reshape(3, B, H, W)
    image = jnp.transpose(image, (1, 0, 2, 3))
    return _inpaint(image)


# speedup vs baseline: 1.6689x; 1.0297x over previous
import functools

import jax
import jax.numpy as jnp
from jax import lax
from jax.experimental import pallas as pl
from jax.experimental.pallas import tpu as pltpu
from jax.experimental.pallas import tpu_sc as plsc

B, N = 2, 262144
H, W = 384, 384
KS = 5
PAD = KS // 2
TOTAL = B * H * W

HW = H * W
NTILES = 16
PPT = HW // NTILES
CP = 16384
NB = N // CP
IMAX = jnp.iinfo(jnp.int32).max


def _ce(a, b):
    return jnp.minimum(a, b), jnp.maximum(a, b)


def _median25(w):
    work = list(w[:14])
    nxt = 14
    while len(work) > 3:
        s = len(work)
        for i in range(s - 1):
            work[i], work[i + 1] = _ce(work[i], work[i + 1])
        for i in range(s - 2, 0, -1):
            work[i - 1], work[i] = _ce(work[i - 1], work[i])
        work = work[1:-1]
        work.append(w[nxt])
        nxt += 1
    a, b, c = work
    return jnp.maximum(jnp.minimum(a, b), jnp.minimum(jnp.maximum(a, b), c))


def _reflect_pad2(x):
    x = jnp.concatenate(
        [x[2:3, :], x[1:2, :], x, x[H - 2:H - 1, :], x[H - 3:H - 2, :]], axis=0)
    x = jnp.concatenate(
        [x[:, 2:3], x[:, 1:2], x, x[:, W - 2:W - 1], x[:, W - 3:W - 2]], axis=1)
    return x


def _inpaint_kernel(img_ref, out_ref):
    hole = img_ref[0, 0] <= 0.0

    def one_round(_, chans):
        outs = []
        for c in chans:
            p = _reflect_pad2(c)
            wins = [p[i:i + H, j:j + W] for i in range(KS) for j in range(KS)]
            med = _median25(wins)
            outs.append(jnp.where(hole, med, c))
        return tuple(outs)

    chans = tuple(img_ref[0, c] for c in range(3))
    chans = lax.fori_loop(0, 10, one_round, chans, unroll=False)

    s = ((chans[0] > 0.0) | (chans[1] > 0.0) | (chans[2] > 0.0)).astype(jnp.float32)
    sp = jnp.pad(s, ((1, 1), (1, 1)))
    neigh = sum(sp[i:i + H, j:j + W] for i in range(3) for j in range(3))
    keep = (neigh >= 6.0).astype(jnp.float32)
    for c in range(3):
        out_ref[0, c] = chans[c] * keep


def _inpaint(image):
    return pl.pallas_call(
        _inpaint_kernel,
        grid=(B,),
        in_specs=[pl.BlockSpec((1, 3, H, W), lambda b: (b, 0, 0, 0))],
        out_specs=pl.BlockSpec((1, 3, H, W), lambda b: (b, 0, 0, 0)),
        out_shape=jax.ShapeDtypeStruct((B, 3, H, W), jnp.float32),
    )(image)


def _bf(x):
    return x.astype(jnp.bfloat16).astype(jnp.float32)


def _proj_kernel(K_ref, T_ref, cloud_ref, lin_ref, dep_ref):
    b = pl.program_id(0)
    c0, c1, c2, c3 = (_bf(cloud_ref[0, i]) for i in range(4))
    cc = [(_bf(T_ref[0, i, 0]) * c0 + _bf(T_ref[0, i, 1]) * c1)
          + (_bf(T_ref[0, i, 2]) * c2 + _bf(T_ref[0, i, 3]) * c3)
          for i in range(3)]
    ccb = [_bf(x) for x in cc]
    pr = [(_bf(K_ref[0, i, 0]) * ccb[0] + _bf(K_ref[0, i, 1]) * ccb[1])
          + _bf(K_ref[0, i, 2]) * ccb[2] for i in range(3)]
    u = jnp.clip(pr[0] / pr[2], 0.0, W - 1).astype(jnp.int32)
    v = jnp.clip(pr[1] / pr[2], 0.0, H - 1).astype(jnp.int32)
    lin_ref[0] = b * HW + v * W + u
    dep_ref[0] = cc[2].astype(jnp.int32)


def _project(cloud, K, T):
    c4 = cloud.reshape(B, 4, 512, 512)
    lin, dep = pl.pallas_call(
        _proj_kernel,
        grid=(B,),
        in_specs=[
            pl.BlockSpec((1, 3, 3), lambda b: (b, 0, 0), memory_space=pltpu.SMEM),
            pl.BlockSpec((1, 4, 4), lambda b: (b, 0, 0), memory_space=pltpu.SMEM),
            pl.BlockSpec((1, 4, 512, 512), lambda b: (b, 0, 0, 0)),
        ],
        out_specs=[pl.BlockSpec((1, 512, 512), lambda b: (b, 0, 0))] * 2,
        out_shape=[jax.ShapeDtypeStruct((B, 512, 512), jnp.int32)] * 2,
    )(K, T, c4)
    return lin.reshape(-1), dep.reshape(-1)


def _any16(m):
    return plsc.all_reduce_population_count(m)[0] > 0


def _zbuf_body(lin_hbm, dep_hbm, r0_hbm, r1_hbm, r2_hbm,
               img0_hbm, img1_hbm, img2_hbm,
               idxv, dv, minbuf, idbuf, idsafe, g0, g1, g2, sem):
    cid = lax.axis_index("c")
    sid = lax.axis_index("s")
    lo = cid * HW + sid * PPT
    hi = lo + PPT
    pt_base = cid * N

    def init_bufs(k, _):
        minbuf[pl.ds(k * 16, 16)] = jnp.full((16,), IMAX, jnp.int32)
        idbuf[pl.ds(k * 16, 16)] = jnp.full((16,), -1, jnp.int32)
        return 0
    lax.fori_loop(0, PPT // 16, init_bufs, 0)

    GS = 8

    def blk_body(blk, _):
        off = pt_base + blk * CP
        pltpu.sync_copy(lin_hbm.at[pl.ds(off, CP)], idxv)
        pltpu.sync_copy(dep_hbm.at[pl.ds(off, CP)], dv)

        def chunk(i, _):
            base = i * (16 * GS)
            idxs = [idxv[pl.ds(base + k * 16, 16)] for k in range(GS)]
            inms = [(ix >= lo) & (ix < hi) for ix in idxs]
            offs = [jnp.clip(ix - lo, 0, PPT - 1) for ix in idxs]
            ds_ = [dv[pl.ds(base + k * 16, 16)] for k in range(GS)]
            gids = [pt_base + blk * CP + base + k * 16
                    + jnp.arange(16, dtype=jnp.int32) for k in range(GS)]
            betters = []
            for k in range(GS):
                c_d = plsc.load_gather(minbuf, [offs[k]], mask=inms[k])
                c_i = plsc.load_gather(idbuf, [offs[k]], mask=inms[k])
                betters.append(inms[k] & ((ds_[k] < c_d)
                                          | ((ds_[k] == c_d) & (gids[k] > c_i))))

            def cond(nds):
                m = nds[0]
                for k in range(1, GS):
                    m = m | nds[k]
                return _any16(m)

            def body(nds):
                out = []
                for k in range(GS):
                    nd = nds[k]
                    plsc.store_scatter(minbuf, [offs[k]], ds_[k], mask=nd)
                    c_d = plsc.load_gather(minbuf, [offs[k]], mask=nd)
                    idw = nd & (ds_[k] == c_d)
                    plsc.store_scatter(idbuf, [offs[k]], gids[k], mask=idw)
                    c_i = plsc.load_gather(idbuf, [offs[k]], mask=nd)
                    out.append(nd & ((ds_[k] < c_d)
                                     | ((ds_[k] == c_d) & (gids[k] > c_i))))
                return tuple(out)
            lax.while_loop(cond, body, tuple(betters))
            return 0
        lax.fori_loop(0, CP // (16 * GS), chunk, 0)
        return 0
    lax.fori_loop(0, NB, blk_body, 0)

    def mk_safe(k, _):
        ids = idbuf[pl.ds(k * 16, 16)]
        spread = pt_base + k * 16 + jnp.arange(16, dtype=jnp.int32)
        idsafe[pl.ds(k * 16, 16)] = jnp.where(ids >= 0, ids, spread)
        return 0
    lax.fori_loop(0, PPT // 16, mk_safe, 0)

    def gblk(gi, _):
        base = gi * 128
        h0 = pltpu.async_copy(r0_hbm.at[idsafe.at[pl.ds(base, 128)]],
                              g0.at[pl.ds(base, 128)], sem)
        h1 = pltpu.async_copy(r1_hbm.at[idsafe.at[pl.ds(base, 128)]],
                              g1.at[pl.ds(base, 128)], sem)
        h2 = pltpu.async_copy(r2_hbm.at[idsafe.at[pl.ds(base, 128)]],
                              g2.at[pl.ds(base, 128)], sem)
        h0.wait()
        h1.wait()
        h2.wait()
        return 0
    lax.fori_loop(0, PPT // 128, gblk, 0)

    def fin(k, _):
        sl = pl.ds(k * 16, 16)
        valid = idbuf[sl] >= 0
        g0[sl] = jnp.where(valid, g0[sl], -0.001)
        g1[sl] = jnp.where(valid, g1[sl], -0.001)
        g2[sl] = jnp.where(valid, g2[sl], -0.001)
        return 0
    lax.fori_loop(0, PPT // 16, fin, 0)

    pltpu.sync_copy(g0, img0_hbm.at[pl.ds(lo, PPT)])
    pltpu.sync_copy(g1, img1_hbm.at[pl.ds(lo, PPT)])
    pltpu.sync_copy(g2, img2_hbm.at[pl.ds(lo, PPT)])


def _zbuffer(lin, dep, r0, r1, r2):
    run = pl.kernel(
        _zbuf_body,
        out_type=[jax.ShapeDtypeStruct((TOTAL,), jnp.float32)] * 3,
        mesh=plsc.VectorSubcoreMesh(core_axis_name="c", subcore_axis_name="s"),
        compiler_params=pltpu.CompilerParams(needs_layout_passes=False),
        scratch_types=[
            pltpu.VMEM((CP,), jnp.int32),
            pltpu.VMEM((CP,), jnp.int32),
            pltpu.VMEM((PPT,), jnp.int32),
            pltpu.VMEM((PPT,), jnp.int32),
            pltpu.VMEM((PPT,), jnp.int32),
            pltpu.VMEM((PPT,), jnp.float32),
            pltpu.VMEM((PPT,), jnp.float32),
            pltpu.VMEM((PPT,), jnp.float32),
            pltpu.SemaphoreType.DMA,
        ],
    )
    return run(lin, dep, r0, r1, r2)


def kernel(cloud, rgb_vec, K, T):
    lin, dep = _project(cloud, K, T)
    rgbp = jnp.transpose(rgb_vec, (1, 0, 2)).reshape(3, -1)
    img0, img1, img2 = _zbuffer(lin, dep, rgbp[0], rgbp[1], rgbp[2])
    image = jnp.stack([img0, img1, img2], 0).reshape(3, B, H, W)
    image = jnp.transpose(image, (1, 0, 2, 3))
    return _inpaint(image)
